# baseline (speedup 1.0000x reference)
import functools

import jax
import jax.numpy as jnp
from jax import lax
from jax.experimental import pallas as pl
from jax.experimental.pallas import tpu as pltpu
from jax.experimental.pallas import tpu_sc as plsc

D = 128
NCORES = 2
NSUB = 16
BN_SCALE = float(1.0 / (1.0 + 1e-5) ** 0.5)


def _mm_body(n_out, x_ref, w_ref, b_ref, *out_refs):
    acc = lax.dot_general(x_ref[...], w_ref[...], (((1,), (0,)), ((), ())),
                          preferred_element_type=jnp.float32)
    acc = acc + b_ref[...]
    for j in range(n_out):
        out_refs[j][...] = acc[:, j * D:(j + 1) * D]


def _mm_multi(x, w, b, n_out, bm):
    nv = x.shape[0]
    assert nv % bm == 0
    grid = (nv // bm,)
    return pl.pallas_call(
        functools.partial(_mm_body, n_out),
        grid=grid,
        in_specs=[
            pl.BlockSpec((bm, D), lambda i: (i, 0)),
            pl.BlockSpec((D, n_out * D), lambda i: (0, 0)),
            pl.BlockSpec((1, n_out * D), lambda i: (0, 0)),
        ],
        out_specs=[pl.BlockSpec((bm, D), lambda i: (i, 0))] * n_out,
        out_shape=[jax.ShapeDtypeStruct((nv, D), jnp.float32)] * n_out,
    )(x, w, b.reshape(1, -1))


def _node_update_body(x_ref, s_ref, h_ref, gb_ref, o_ref):
    z = (s_ref[...] + h_ref[...]) * gb_ref[0:1, :] + gb_ref[1:2, :]
    o_ref[...] = x_ref[...] + z / (1.0 + jnp.exp(-z))


def _node_update(x, s, h, gb, bm):
    nv = x.shape[0]
    assert nv % bm == 0
    return pl.pallas_call(
        _node_update_body,
        grid=(nv // bm,),
        in_specs=[
            pl.BlockSpec((bm, D), lambda i: (i, 0)),
            pl.BlockSpec((bm, D), lambda i: (i, 0)),
            pl.BlockSpec((bm, D), lambda i: (i, 0)),
            pl.BlockSpec((2, D), lambda i: (0, 0)),
        ],
        out_specs=pl.BlockSpec((bm, D), lambda i: (i, 0)),
        out_shape=jax.ShapeDtypeStruct((nv, D), jnp.float32),
    )(x, s, h, gb)


def _edge_pass(a_t, b_t, c_t, y, src, dst, gb):
    te = c_t.shape[0]
    nw = NCORES * NSUB
    pw = te // nw
    kb = 200
    assert te % nw == 0 and pw % kb == 0
    mesh = plsc.VectorSubcoreMesh(core_axis_name="c", subcore_axis_name="s",
                                  num_cores=NCORES, num_subcores=NSUB)

    @functools.partial(
        pl.kernel, mesh=mesh,
        out_type=(jax.ShapeDtypeStruct((te, D), jnp.float32),
                  jax.ShapeDtypeStruct((te, D), jnp.float32)),
        scratch_types=[
            pltpu.VMEM((kb,), jnp.int32),
            pltpu.VMEM((kb,), jnp.int32),
            pltpu.VMEM((kb, D), jnp.float32),
            pltpu.VMEM((kb, D), jnp.float32),
            pltpu.VMEM((kb, D), jnp.float32),
            pltpu.VMEM((kb, D), jnp.float32),
            pltpu.VMEM((2 * D,), jnp.float32),
            pltpu.SemaphoreType.DMA,
            pltpu.SemaphoreType.DMA,
        ],
    )
    def body(a_hbm, b_hbm, c_hbm, y_hbm, src_hbm, dst_hbm, gb_hbm,
             sig_out, y_out, sidx, didx, abuf, bbuf, cbuf, ybuf, gbuf,
             sem1, sem2):
        cc = lax.axis_index("c")
        ss = lax.axis_index("s")
        wbase = (ss * NCORES + cc) * pw
        pltpu.sync_copy(gb_hbm, gbuf)
        gvs = [gbuf[pl.ds(k * 16, 16)] for k in range(16)]

        @pl.loop(0, pw // kb)
        def _(i):
            base = wbase + i * kb
            pltpu.sync_copy(src_hbm.at[pl.ds(base, kb)], sidx)
            pltpu.sync_copy(dst_hbm.at[pl.ds(base, kb)], didx)
            cp_a = pltpu.async_copy(a_hbm.at[sidx], abuf, sem1)
            cp_b = pltpu.async_copy(b_hbm.at[didx], bbuf, sem2)
            pltpu.sync_copy(c_hbm.at[pl.ds(base, kb)], cbuf)
            pltpu.sync_copy(y_hbm.at[pl.ds(base, kb)], ybuf)
            cp_a.wait()
            cp_b.wait()

            @pl.loop(0, kb)
            def _(r):
                for k in range(8):
                    sl = pl.ds(k * 16, 16)
                    m = abuf[r, sl] + bbuf[r, sl] + cbuf[r, sl]
                    abuf[r, sl] = 1.0 / (1.0 + jnp.exp(-m))
                    z = m * gvs[k] + gvs[k + 8]
                    bbuf[r, sl] = ybuf[r, sl] + z / (1.0 + jnp.exp(-z))

            pltpu.sync_copy(abuf, sig_out.at[pl.ds(base, kb)])
            pltpu.sync_copy(bbuf, y_out.at[pl.ds(base, kb)])

    return body(a_t, b_t, c_t, y, src, dst, gb)


def _aggregate(sig, u_t, src, dst, nv, ch):
    te = src.shape[0]
    pt = te // NSUB
    seg = 2000
    nseg = pt // seg
    stripe = ch // NSUB
    assert pt % seg == 0 and seg % 16 == 0 and stripe % 8 == 0
    rounds = -(-nv // (2 * ch))
    nvpad = rounds * 2 * ch
    arows = stripe + 8
    lcap = -(-(seg + 1) // 64) * 64
    qcap = lcap + 128
    shift = stripe.bit_length() - 1
    assert stripe == (1 << shift)
    mesh = plsc.VectorSubcoreMesh(core_axis_name="c", subcore_axis_name="s",
                                  num_cores=NCORES, num_subcores=NSUB)

    @functools.partial(
        pl.kernel, mesh=mesh,
        compiler_params=pltpu.CompilerParams(needs_layout_passes=False),
        out_type=jax.ShapeDtypeStruct((nvpad, D), jnp.float32),
        scratch_types=[
            pltpu.VMEM_SHARED((2 * NSUB * 3 * lcap,), jnp.int32),
            pltpu.VMEM_SHARED((2 * NSUB * 16,), jnp.int32),
            pltpu.VMEM((arows, 2 * D), jnp.float32),
            pltpu.VMEM((seg,), jnp.int32),
            pltpu.VMEM((seg,), jnp.int32),
            pltpu.VMEM((lcap,), jnp.int32),
            pltpu.VMEM((lcap,), jnp.int32),
            pltpu.VMEM((lcap,), jnp.int32),
            pltpu.VMEM((256,), jnp.int32),
            pltpu.VMEM((256,), jnp.int32),
            pltpu.VMEM((256,), jnp.int32),
            pltpu.VMEM((qcap,), jnp.int32),
            pltpu.VMEM((qcap,), jnp.int32),
            pltpu.VMEM((qcap,), jnp.int32),
            pltpu.VMEM((64, D), jnp.float32),
            pltpu.VMEM((64, D), jnp.float32),
            pltpu.VMEM((32, D), jnp.float32),
            pltpu.VMEM((16,), jnp.int32),
            pltpu.VMEM((NSUB * 16,), jnp.int32),
            pltpu.SemaphoreType.DMA,
            pltpu.SemaphoreType.DMA,
            pltpu.SemaphoreType.DMA,
            pltpu.SemaphoreType.DMA,
        ],
    )
    def body(sig_hbm, u_hbm, src_hbm, dst_hbm, h_out,
             lists, counts, acc, dseg, sseg, tq, sq, dq, lt, ls, ldl,
             tq2, sq2, dq2, sigb, ub, hbuf, cntw, cbuf,
             sem1, sem2, sem3, sem4):
        cc = lax.axis_index("c")
        ss = lax.axis_index("s")
        iota = lax.iota(jnp.int32, 16)
        zv = jnp.zeros((16,), jnp.float32)

        def fire_batch(off):
            cp_a = pltpu.async_copy(
                sig_hbm.at[tq2.at[pl.ds(off, 32)]],
                sigb.at[pl.ds(0, 32)], sem1)
            cp_b = pltpu.async_copy(
                u_hbm.at[sq2.at[pl.ds(off, 32)]], ub.at[pl.ds(0, 32)], sem2)
            cp_c = pltpu.async_copy(
                sig_hbm.at[tq2.at[pl.ds(off + 32, 32)]],
                sigb.at[pl.ds(32, 32)], sem3)
            cp_d = pltpu.async_copy(
                u_hbm.at[sq2.at[pl.ds(off + 32, 32)]],
                ub.at[pl.ds(32, 32)], sem4)

            def rmw_half(h):
                @pl.loop(0, 2)
                def _(q):
                    qq = h * 2 + q
                    dlv = dq2[pl.ds(off + qq * 16, 16)]
                    for lane in range(16):
                        rr = qq * 16 + lane
                        dl = dlv[lane]
                        for k in range(8):
                            sl = pl.ds(k * 16, 16)
                            sl2 = pl.ds(D + k * 16, 16)
                            sg = sigb[rr, sl]
                            acc[dl, sl] = acc[dl, sl] + sg * ub[rr, sl]
                            acc[dl, sl2] = acc[dl, sl2] + sg

            cp_a.wait()
            cp_b.wait()
            rmw_half(0)
            cp_c.wait()
            cp_d.wait()
            rmw_half(1)

        def drain(qcnt):
            nfull = qcnt // 64

            def fire_b(b, carry):
                fire_batch(b * 64)
                return carry

            lax.fori_loop(0, nfull, fire_b, jnp.int32(0))
            rem_base = nfull * 64
            for k in range(4):
                slk = pl.ds(k * 16, 16)
                tq2[slk] = tq2[pl.ds(rem_base + k * 16, 16)]
                sq2[slk] = sq2[pl.ds(rem_base + k * 16, 16)]
                dq2[slk] = dq2[pl.ds(rem_base + k * 16, 16)]
            return qcnt - nfull * 64

        @pl.loop(0, rounds)
        def _(r):
            base = (2 * r + cc) * ch

            @pl.loop(0, arows)
            def _(rw):
                for k in range(16):
                    acc[rw, pl.ds(k * 16, 16)] = zv

            def seg_body(g, qcnt):
                segbase = ss * pt + g * seg
                pltpu.sync_copy(dst_hbm.at[pl.ds(segbase, seg)], dseg)
                pltpu.sync_copy(src_hbm.at[pl.ds(segbase, seg)], sseg)

                def scan_body(v, cnt):
                    sl = pl.ds(v * 16, 16)
                    d = dseg[sl]
                    sv = sseg[sl]
                    m = (d >= base) & (d < base + ch)
                    pos = plsc.cumsum(m.astype(jnp.int32))
                    idx = cnt + pos - 1
                    plsc.store_scatter(tq, [idx], segbase + v * 16 + iota,
                                       mask=m)
                    plsc.store_scatter(sq, [idx], sv, mask=m)
                    plsc.store_scatter(dq, [idx], d - base, mask=m)
                    return cnt + jnp.max(pos)

                cnt = lax.fori_loop(0, seg // 16, scan_body, jnp.int32(0))

                par = (g & 1) * NSUB
                def pub(q, carry):
                    slq = pl.ds(q * 256, 256)
                    lb = (par + ss) * 3 * lcap + q * 256
                    pltpu.sync_copy(tq.at[slq], lists.at[pl.ds(lb, 256)])
                    pltpu.sync_copy(sq.at[slq],
                                    lists.at[pl.ds(lb + lcap, 256)])
                    pltpu.sync_copy(dq.at[slq],
                                    lists.at[pl.ds(lb + 2 * lcap, 256)])
                    return carry

                lax.fori_loop(0, (cnt + 255) // 256, pub, jnp.int32(0))
                cntw[pl.ds(0, 16)] = jnp.full((16,), 0, jnp.int32) + cnt
                pltpu.sync_copy(cntw,
                                counts.at[pl.ds((par + ss) * 16, 16)])
                plsc.subcore_barrier()
                pltpu.sync_copy(counts.at[pl.ds(par * 16, NSUB * 16)], cbuf)

                def list_body(j, qcnt):
                    cj = cbuf[pl.ds(j * 16, 16)][0]

                    def chunk_body(q, qcnt):
                        lb = (par + j) * 3 * lcap + q * 256
                        pltpu.sync_copy(lists.at[pl.ds(lb, 256)], lt)
                        pltpu.sync_copy(lists.at[pl.ds(lb + lcap, 256)], ls)
                        pltpu.sync_copy(lists.at[pl.ds(lb + 2 * lcap, 256)],
                                        ldl)
                        nvr = (jnp.minimum(cj - q * 256, 256) + 15) // 16

                        def filt(v, qcnt):
                            sl = pl.ds(v * 16, 16)
                            tv = lt[sl]
                            sv = ls[sl]
                            dl = ldl[sl]
                            gl = q * 256 + v * 16 + iota
                            own = (gl < cj) & ((dl >> shift) == ss)
                            pos = plsc.cumsum(own.astype(jnp.int32))
                            idx = qcnt + pos - 1
                            plsc.store_scatter(tq2, [idx], tv, mask=own)
                            plsc.store_scatter(sq2, [idx], sv, mask=own)
                            plsc.store_scatter(
                                dq2, [idx], dl & (stripe - 1), mask=own)
                            return qcnt + jnp.max(pos)

                        return lax.fori_loop(0, nvr, filt, qcnt)

                    qcnt = lax.fori_loop(0, (cj + 255) // 256, chunk_body,
                                         qcnt)
                    return drain(qcnt)

                qcnt = lax.fori_loop(0, NSUB, list_body, qcnt)
                return qcnt

            qcnt = lax.fori_loop(0, nseg, seg_body, jnp.int32(0))

            padtot = ((qcnt + 63) // 64) * 64
            for k in range(4):
                pidx = qcnt + k * 16 + iota
                pm = pidx < padtot
                pv = k * 16 + iota
                plsc.store_scatter(tq2, [pidx], pv, mask=pm)
                plsc.store_scatter(sq2, [pidx], pv, mask=pm)
                plsc.store_scatter(dq2, [pidx], stripe + (pv & 7), mask=pm)
            qcnt = drain(padtot)

            @pl.loop(0, stripe // 32)
            def _(o):
                off = o * 32

                @pl.loop(0, 32)
                def _(rw):
                    for k in range(8):
                        sl = pl.ds(k * 16, 16)
                        num = acc[off + rw, sl]
                        den = acc[off + rw, pl.ds(D + k * 16, 16)]
                        hbuf[rw, sl] = num / (den + 1e-6)

                pltpu.sync_copy(
                    hbuf, h_out.at[pl.ds(base + ss * stripe + off, 32)])

    return body(sig, u_t, src, dst)


def _gcao_layer(x, y, src, dst, p, ch):
    nv = x.shape[0]
    w4 = jnp.concatenate([p['W_sg'], p['W_dg'], p['W_du'], p['W_su']], axis=1)
    b4 = jnp.concatenate([p['b_sg'], p['b_dg'], p['b_du'], p['b_su']])
    a_t, b_t, u_t, s_t = _mm_multi(x, w4, b4, 4, 1000)
    c_t, = _mm_multi(y, p['W_eg'], p['b_eg'], 1, 2000)
    ge = p['g_edge'] * BN_SCALE
    gbe = jnp.concatenate([ge, p['beta_edge']])
    sig, y_out = _edge_pass(a_t, b_t, c_t, y, src, dst, gbe)
    h = _aggregate(sig, u_t, src, dst, nv, ch)
    gn = jnp.stack([p['g_node'] * BN_SCALE, p['beta_node']])
    x_out = _node_update(x, s_t, h[:nv], gn, 1000)
    return x_out, y_out


def kernel(atom_feats, bond_attr, triplet_feats, edge_index, angle_index,
           params1, params2):
    a_src, a_dst = angle_index[0], angle_index[1]
    e_src, e_dst = edge_index[0], edge_index[1]
    bond_mid, triplet_out = _gcao_layer(bond_attr, triplet_feats,
                                        a_src, a_dst, params1, ch=4096)
    atom_out, bond_out = _gcao_layer(atom_feats, bond_mid,
                                     e_src, e_dst, params2, ch=4096)
    return (atom_out, bond_out, triplet_out)


# speedup vs baseline: 1.3200x; 1.3200x over previous
import functools

import jax
import jax.numpy as jnp
from jax import lax
from jax.experimental import pallas as pl
from jax.experimental.pallas import tpu as pltpu
from jax.experimental.pallas import tpu_sc as plsc

D = 128
NCORES = 2
NSUB = 16
BN_SCALE = float(1.0 / (1.0 + 1e-5) ** 0.5)


def _mm_body(n_out, x_ref, w_ref, b_ref, *out_refs):
    acc = lax.dot_general(x_ref[...], w_ref[...], (((1,), (0,)), ((), ())),
                          preferred_element_type=jnp.float32)
    acc = acc + b_ref[...]
    for j in range(n_out):
        out_refs[j][...] = acc[:, j * D:(j + 1) * D]


def _mm_multi(x, w, b, n_out, bm):
    nv = x.shape[0]
    assert nv % bm == 0
    grid = (nv // bm,)
    return pl.pallas_call(
        functools.partial(_mm_body, n_out),
        grid=grid,
        in_specs=[
            pl.BlockSpec((bm, D), lambda i: (i, 0)),
            pl.BlockSpec((D, n_out * D), lambda i: (0, 0)),
            pl.BlockSpec((1, n_out * D), lambda i: (0, 0)),
        ],
        out_specs=[pl.BlockSpec((bm, D), lambda i: (i, 0))] * n_out,
        out_shape=[jax.ShapeDtypeStruct((nv, D), jnp.float32)] * n_out,
    )(x, w, b.reshape(1, -1))


def _node_update_body(x_ref, s_ref, h_ref, gb_ref, o_ref):
    z = (s_ref[...] + h_ref[...]) * gb_ref[0:1, :] + gb_ref[1:2, :]
    o_ref[...] = x_ref[...] + z / (1.0 + jnp.exp(-z))


def _node_update(x, s, h, gb, bm):
    nv = x.shape[0]
    assert nv % bm == 0
    return pl.pallas_call(
        _node_update_body,
        grid=(nv // bm,),
        in_specs=[
            pl.BlockSpec((bm, D), lambda i: (i, 0)),
            pl.BlockSpec((bm, D), lambda i: (i, 0)),
            pl.BlockSpec((bm, D), lambda i: (i, 0)),
            pl.BlockSpec((2, D), lambda i: (0, 0)),
        ],
        out_specs=pl.BlockSpec((bm, D), lambda i: (i, 0)),
        out_shape=jax.ShapeDtypeStruct((nv, D), jnp.float32),
    )(x, s, h, gb)


def _edge_pass(a_t, b_t, c_t, y, src, dst, gb):
    te = c_t.shape[0]
    nw = NCORES * NSUB
    pw = te // nw
    kb = 200
    assert te % nw == 0 and pw % kb == 0
    mesh = plsc.VectorSubcoreMesh(core_axis_name="c", subcore_axis_name="s",
                                  num_cores=NCORES, num_subcores=NSUB)

    @functools.partial(
        pl.kernel, mesh=mesh,
        out_type=(jax.ShapeDtypeStruct((te, D), jnp.float32),
                  jax.ShapeDtypeStruct((te, D), jnp.float32)),
        scratch_types=[
            pltpu.VMEM((kb,), jnp.int32),
            pltpu.VMEM((kb,), jnp.int32),
            pltpu.VMEM((kb, D), jnp.float32),
            pltpu.VMEM((kb, D), jnp.float32),
            pltpu.VMEM((kb, D), jnp.float32),
            pltpu.VMEM((kb, D), jnp.float32),
            pltpu.VMEM((2 * D,), jnp.float32),
            pltpu.SemaphoreType.DMA,
            pltpu.SemaphoreType.DMA,
        ],
    )
    def body(a_hbm, b_hbm, c_hbm, y_hbm, src_hbm, dst_hbm, gb_hbm,
             sig_out, y_out, sidx, didx, abuf, bbuf, cbuf, ybuf, gbuf,
             sem1, sem2):
        cc = lax.axis_index("c")
        ss = lax.axis_index("s")
        wbase = (ss * NCORES + cc) * pw
        pltpu.sync_copy(gb_hbm, gbuf)
        gvs = [gbuf[pl.ds(k * 16, 16)] for k in range(16)]

        @pl.loop(0, pw // kb)
        def _(i):
            base = wbase + i * kb
            pltpu.sync_copy(src_hbm.at[pl.ds(base, kb)], sidx)
            pltpu.sync_copy(dst_hbm.at[pl.ds(base, kb)], didx)
            cp_a = pltpu.async_copy(a_hbm.at[sidx], abuf, sem1)
            cp_b = pltpu.async_copy(b_hbm.at[didx], bbuf, sem2)
            pltpu.sync_copy(c_hbm.at[pl.ds(base, kb)], cbuf)
            pltpu.sync_copy(y_hbm.at[pl.ds(base, kb)], ybuf)
            cp_a.wait()
            cp_b.wait()

            @pl.loop(0, kb)
            def _(r):
                for k in range(8):
                    sl = pl.ds(k * 16, 16)
                    m = abuf[r, sl] + bbuf[r, sl] + cbuf[r, sl]
                    abuf[r, sl] = 1.0 / (1.0 + jnp.exp(-m))
                    z = m * gvs[k] + gvs[k + 8]
                    bbuf[r, sl] = ybuf[r, sl] + z / (1.0 + jnp.exp(-z))

            pltpu.sync_copy(abuf, sig_out.at[pl.ds(base, kb)])
            pltpu.sync_copy(bbuf, y_out.at[pl.ds(base, kb)])

    return body(a_t, b_t, c_t, y, src, dst, gb)


def _aggregate(sig, u_t, src, dst, nv, ch):
    te = src.shape[0]
    pt = te // NSUB
    seg = 2000
    nseg = pt // seg
    stripe = ch // NSUB
    assert pt % seg == 0 and seg % 16 == 0 and stripe % 8 == 0
    rounds = -(-nv // (2 * ch))
    nvpad = rounds * 2 * ch
    arows = stripe + 8
    lcap = -(-(seg + 1) // 64) * 64
    qcap = 512 + 128
    shift = stripe.bit_length() - 1
    assert stripe == (1 << shift)
    W = NSUB * lcap
    mesh = plsc.VectorSubcoreMesh(core_axis_name="c", subcore_axis_name="s",
                                  num_cores=NCORES, num_subcores=NSUB)

    @functools.partial(
        pl.kernel, mesh=mesh,
        compiler_params=pltpu.CompilerParams(needs_layout_passes=False),
        out_type=jax.ShapeDtypeStruct((nvpad, D), jnp.float32),
        scratch_types=[
            pltpu.VMEM_SHARED((2 * 3 * NSUB * lcap,), jnp.int32),
            pltpu.VMEM_SHARED((2 * NSUB * 16,), jnp.int32),
            pltpu.VMEM((arows, 2 * D), jnp.float32),
            pltpu.VMEM((seg,), jnp.int32),
            pltpu.VMEM((seg,), jnp.int32),
            pltpu.VMEM((lcap,), jnp.int32),
            pltpu.VMEM((lcap,), jnp.int32),
            pltpu.VMEM((lcap,), jnp.int32),
            pltpu.VMEM((512,), jnp.int32),
            pltpu.VMEM((512,), jnp.int32),
            pltpu.VMEM((512,), jnp.int32),
            pltpu.VMEM((qcap,), jnp.int32),
            pltpu.VMEM((qcap,), jnp.int32),
            pltpu.VMEM((qcap,), jnp.int32),
            pltpu.VMEM((64, D), jnp.float32),
            pltpu.VMEM((64, D), jnp.float32),
            pltpu.VMEM((32, D), jnp.float32),
            pltpu.VMEM((16,), jnp.int32),
            pltpu.VMEM((NSUB * 16,), jnp.int32),
            pltpu.SemaphoreType.DMA,
            pltpu.SemaphoreType.DMA,
            pltpu.SemaphoreType.DMA,
            pltpu.SemaphoreType.DMA,
        ],
    )
    def body(sig_hbm, u_hbm, src_hbm, dst_hbm, h_out,
             lists, counts, acc, dseg, sseg, tq, sq, dq, lt, ls, ldl,
             tq2, sq2, dq2, sigb, ub, hbuf, cntw, cbuf,
             sem1, sem2, sem3, sem4):
        cc = lax.axis_index("c")
        ss = lax.axis_index("s")
        iota = lax.iota(jnp.int32, 16)
        zv = jnp.zeros((16,), jnp.float32)

        def fire_batch(off):
            cp_s = pltpu.async_copy(
                sig_hbm.at[tq2.at[pl.ds(off, 64)]], sigb, sem1)
            cp_u = pltpu.async_copy(
                u_hbm.at[sq2.at[pl.ds(off, 64)]], ub, sem2)
            cp_s.wait()
            cp_u.wait()

            @pl.loop(0, 4)
            def _(q):
                dlv = dq2[pl.ds(off + q * 16, 16)]
                for lane in range(16):
                    rr = q * 16 + lane
                    dl = dlv[lane]
                    for k in range(8):
                        sl = pl.ds(k * 16, 16)
                        sl2 = pl.ds(D + k * 16, 16)
                        sg = sigb[rr, sl]
                        acc[dl, sl] = acc[dl, sl] + sg * ub[rr, sl]
                        acc[dl, sl2] = acc[dl, sl2] + sg

        def drain(qcnt):
            nfull = qcnt // 64

            def fire_b(b, carry):
                fire_batch(b * 64)
                return carry

            lax.fori_loop(0, nfull, fire_b, jnp.int32(0))
            rem_base = nfull * 64
            for k in range(4):
                slk = pl.ds(k * 16, 16)
                tq2[slk] = tq2[pl.ds(rem_base + k * 16, 16)]
                sq2[slk] = sq2[pl.ds(rem_base + k * 16, 16)]
                dq2[slk] = dq2[pl.ds(rem_base + k * 16, 16)]
            return qcnt - nfull * 64

        @pl.loop(0, rounds)
        def _(r):
            base = (2 * r + cc) * ch

            @pl.loop(0, arows)
            def _(rw):
                for k in range(16):
                    acc[rw, pl.ds(k * 16, 16)] = zv

            def seg_body(g, qcnt):
                segbase = ss * pt + g * seg
                pltpu.sync_copy(dst_hbm.at[pl.ds(segbase, seg)], dseg)
                pltpu.sync_copy(src_hbm.at[pl.ds(segbase, seg)], sseg)

                def scan_body(v, cnt):
                    sl = pl.ds(v * 16, 16)
                    d = dseg[sl]
                    sv = sseg[sl]
                    m = (d >= base) & (d < base + ch)
                    pos = plsc.cumsum(m.astype(jnp.int32))
                    idx = cnt + pos - 1
                    plsc.store_scatter(tq, [idx], segbase + v * 16 + iota,
                                       mask=m)
                    plsc.store_scatter(sq, [idx], sv, mask=m)
                    plsc.store_scatter(dq, [idx], d - base, mask=m)
                    return cnt + jnp.max(pos)

                cnt = lax.fori_loop(0, seg // 16, scan_body, jnp.int32(0))

                cnt64 = ((cnt + 63) // 64) * 64
                for k in range(4):
                    sp = cnt + k * 16 + iota
                    plsc.store_scatter(dq, [sp], jnp.full(
                        (16,), NSUB * stripe, jnp.int32), mask=sp < cnt64)

                par = (g & 1)
                cntw[pl.ds(0, 16)] = jnp.full((16,), 0, jnp.int32) + cnt64
                pltpu.sync_copy(
                    cntw, counts.at[pl.ds((par * NSUB + ss) * 16, 16)])
                plsc.subcore_barrier()
                pltpu.sync_copy(
                    counts.at[pl.ds(par * NSUB * 16, NSUB * 16)], cbuf)
                cnts = plsc.load_gather(cbuf, [iota * 16])
                incl = plsc.cumsum(cnts)
                excl = incl - cnts
                myoff = pl.multiple_of(
                    jnp.sum(jnp.where(iota == ss, excl, 0)), 64)
                total = jnp.max(incl)
                lb0 = par * 3 * W

                def pub(q, carry):
                    slq = pl.ds(q * 64, 64)
                    lb = lb0 + myoff + q * 64
                    pltpu.sync_copy(tq.at[slq], lists.at[pl.ds(lb, 64)])
                    pltpu.sync_copy(sq.at[slq], lists.at[pl.ds(lb + W, 64)])
                    pltpu.sync_copy(dq.at[slq],
                                    lists.at[pl.ds(lb + 2 * W, 64)])
                    return carry

                lax.fori_loop(0, cnt64 // 64, pub, jnp.int32(0))
                plsc.subcore_barrier()

                def chunk_body(q, qcnt):
                    lb = lb0 + q * 512
                    pltpu.sync_copy(lists.at[pl.ds(lb, 512)], lt)
                    pltpu.sync_copy(lists.at[pl.ds(lb + W, 512)], ls)
                    pltpu.sync_copy(lists.at[pl.ds(lb + 2 * W, 512)], ldl)
                    nvr = (jnp.minimum(total - q * 512, 512) + 15) // 16

                    def filt(v, qcnt):
                        sl = pl.ds(v * 16, 16)
                        tv = lt[sl]
                        sv = ls[sl]
                        dl = ldl[sl]
                        gl = q * 512 + v * 16 + iota
                        own = (gl < total) & ((dl >> shift) == ss)
                        pos = plsc.cumsum(own.astype(jnp.int32))
                        idx = qcnt + pos - 1
                        plsc.store_scatter(tq2, [idx], tv, mask=own)
                        plsc.store_scatter(sq2, [idx], sv, mask=own)
                        plsc.store_scatter(
                            dq2, [idx], dl & (stripe - 1), mask=own)
                        return qcnt + jnp.max(pos)

                    return drain(lax.fori_loop(0, nvr, filt, qcnt))

                qcnt = lax.fori_loop(0, (total + 511) // 512, chunk_body,
                                     qcnt)
                return qcnt

            qcnt = lax.fori_loop(0, nseg, seg_body, jnp.int32(0))

            padtot = ((qcnt + 63) // 64) * 64
            for k in range(4):
                pidx = qcnt + k * 16 + iota
                pm = pidx < padtot
                pv = k * 16 + iota
                plsc.store_scatter(tq2, [pidx], pv, mask=pm)
                plsc.store_scatter(sq2, [pidx], pv, mask=pm)
                plsc.store_scatter(dq2, [pidx], stripe + (pv & 7), mask=pm)
            qcnt = drain(padtot)

            @pl.loop(0, stripe // 32)
            def _(o):
                off = o * 32

                @pl.loop(0, 32)
                def _(rw):
                    for k in range(8):
                        sl = pl.ds(k * 16, 16)
                        num = acc[off + rw, sl]
                        den = acc[off + rw, pl.ds(D + k * 16, 16)]
                        hbuf[rw, sl] = num / (den + 1e-6)

                pltpu.sync_copy(
                    hbuf, h_out.at[pl.ds(base + ss * stripe + off, 32)])

    return body(sig, u_t, src, dst)


def _gcao_layer(x, y, src, dst, p, ch):
    nv = x.shape[0]
    w4 = jnp.concatenate([p['W_sg'], p['W_dg'], p['W_du'], p['W_su']], axis=1)
    b4 = jnp.concatenate([p['b_sg'], p['b_dg'], p['b_du'], p['b_su']])
    a_t, b_t, u_t, s_t = _mm_multi(x, w4, b4, 4, 1000)
    c_t, = _mm_multi(y, p['W_eg'], p['b_eg'], 1, 2000)
    ge = p['g_edge'] * BN_SCALE
    gbe = jnp.concatenate([ge, p['beta_edge']])
    sig, y_out = _edge_pass(a_t, b_t, c_t, y, src, dst, gbe)
    h = _aggregate(sig, u_t, src, dst, nv, ch)
    gn = jnp.stack([p['g_node'] * BN_SCALE, p['beta_node']])
    x_out = _node_update(x, s_t, h[:nv], gn, 1000)
    return x_out, y_out


def kernel(atom_feats, bond_attr, triplet_feats, edge_index, angle_index,
           params1, params2):
    a_src, a_dst = angle_index[0], angle_index[1]
    e_src, e_dst = edge_index[0], edge_index[1]
    bond_mid, triplet_out = _gcao_layer(bond_attr, triplet_feats,
                                        a_src, a_dst, params1, ch=4096)
    atom_out, bond_out = _gcao_layer(atom_feats, bond_mid,
                                     e_src, e_dst, params2, ch=4096)
    return (atom_out, bond_out, triplet_out)


# speedup vs baseline: 1.4250x; 1.0795x over previous
import functools

import jax
import jax.numpy as jnp
from jax import lax
from jax.experimental import pallas as pl
from jax.experimental.pallas import tpu as pltpu
from jax.experimental.pallas import tpu_sc as plsc

D = 128
NCORES = 2
NSUB = 16
BN_SCALE = float(1.0 / (1.0 + 1e-5) ** 0.5)


def _mm_body(n_out, x_ref, w_ref, b_ref, *out_refs):
    acc = lax.dot_general(x_ref[...], w_ref[...], (((1,), (0,)), ((), ())),
                          preferred_element_type=jnp.float32)
    acc = acc + b_ref[...]
    for j in range(n_out):
        out_refs[j][...] = acc[:, j * D:(j + 1) * D]


def _mm_multi(x, w, b, n_out, bm):
    nv = x.shape[0]
    assert nv % bm == 0
    grid = (nv // bm,)
    return pl.pallas_call(
        functools.partial(_mm_body, n_out),
        grid=grid,
        in_specs=[
            pl.BlockSpec((bm, D), lambda i: (i, 0)),
            pl.BlockSpec((D, n_out * D), lambda i: (0, 0)),
            pl.BlockSpec((1, n_out * D), lambda i: (0, 0)),
        ],
        out_specs=[pl.BlockSpec((bm, D), lambda i: (i, 0))] * n_out,
        out_shape=[jax.ShapeDtypeStruct((nv, D), jnp.float32)] * n_out,
    )(x, w, b.reshape(1, -1))


def _node_update_body(x_ref, s_ref, h_ref, gb_ref, o_ref):
    z = (s_ref[...] + h_ref[...]) * gb_ref[0:1, :] + gb_ref[1:2, :]
    o_ref[...] = x_ref[...] + z / (1.0 + jnp.exp(-z))


def _node_update(x, s, h, gb, bm):
    nv = x.shape[0]
    assert nv % bm == 0
    return pl.pallas_call(
        _node_update_body,
        grid=(nv // bm,),
        in_specs=[
            pl.BlockSpec((bm, D), lambda i: (i, 0)),
            pl.BlockSpec((bm, D), lambda i: (i, 0)),
            pl.BlockSpec((bm, D), lambda i: (i, 0)),
            pl.BlockSpec((2, D), lambda i: (0, 0)),
        ],
        out_specs=pl.BlockSpec((bm, D), lambda i: (i, 0)),
        out_shape=jax.ShapeDtypeStruct((nv, D), jnp.float32),
    )(x, s, h, gb)


def _edge_pass(a_t, b_t, c_t, y, src, dst, gb):
    te = c_t.shape[0]
    nw = NCORES * NSUB
    pw = te // nw
    kb = 200
    assert te % nw == 0 and pw % kb == 0
    mesh = plsc.VectorSubcoreMesh(core_axis_name="c", subcore_axis_name="s",
                                  num_cores=NCORES, num_subcores=NSUB)

    @functools.partial(
        pl.kernel, mesh=mesh,
        out_type=(jax.ShapeDtypeStruct((te, D), jnp.float32),
                  jax.ShapeDtypeStruct((te, D), jnp.float32)),
        scratch_types=[
            pltpu.VMEM((kb,), jnp.int32),
            pltpu.VMEM((kb,), jnp.int32),
            pltpu.VMEM((kb, D), jnp.float32),
            pltpu.VMEM((kb, D), jnp.float32),
            pltpu.VMEM((kb, D), jnp.float32),
            pltpu.VMEM((kb, D), jnp.float32),
            pltpu.VMEM((2 * D,), jnp.float32),
            pltpu.SemaphoreType.DMA,
            pltpu.SemaphoreType.DMA,
        ],
    )
    def body(a_hbm, b_hbm, c_hbm, y_hbm, src_hbm, dst_hbm, gb_hbm,
             sig_out, y_out, sidx, didx, abuf, bbuf, cbuf, ybuf, gbuf,
             sem1, sem2):
        cc = lax.axis_index("c")
        ss = lax.axis_index("s")
        wbase = (ss * NCORES + cc) * pw
        pltpu.sync_copy(gb_hbm, gbuf)
        gvs = [gbuf[pl.ds(k * 16, 16)] for k in range(16)]

        @pl.loop(0, pw // kb)
        def _(i):
            base = wbase + i * kb
            pltpu.sync_copy(src_hbm.at[pl.ds(base, kb)], sidx)
            pltpu.sync_copy(dst_hbm.at[pl.ds(base, kb)], didx)
            cp_a = pltpu.async_copy(a_hbm.at[sidx], abuf, sem1)
            cp_b = pltpu.async_copy(b_hbm.at[didx], bbuf, sem2)
            pltpu.sync_copy(c_hbm.at[pl.ds(base, kb)], cbuf)
            pltpu.sync_copy(y_hbm.at[pl.ds(base, kb)], ybuf)
            cp_a.wait()
            cp_b.wait()

            @pl.loop(0, kb)
            def _(r):
                for k in range(8):
                    sl = pl.ds(k * 16, 16)
                    m = abuf[r, sl] + bbuf[r, sl] + cbuf[r, sl]
                    abuf[r, sl] = 1.0 / (1.0 + jnp.exp(-m))
                    z = m * gvs[k] + gvs[k + 8]
                    bbuf[r, sl] = ybuf[r, sl] + z / (1.0 + jnp.exp(-z))

            pltpu.sync_copy(abuf, sig_out.at[pl.ds(base, kb)])
            pltpu.sync_copy(bbuf, y_out.at[pl.ds(base, kb)])

    return body(a_t, b_t, c_t, y, src, dst, gb)


def _aggregate(sig, u_t, src, dst, nv, ch):
    te = src.shape[0]
    pt = te // NSUB
    seg = 2000
    nseg = pt // seg
    stripe = ch // NSUB
    assert pt % seg == 0 and seg % 16 == 0 and stripe % 8 == 0
    rounds = -(-nv // (2 * ch))
    nvpad = rounds * 2 * ch
    arows = stripe + 8
    lcap = -(-(seg + 1) // 64) * 64
    qcap = 1024 + 128
    shift = stripe.bit_length() - 1
    assert stripe == (1 << shift)
    W = NSUB * lcap
    mesh = plsc.VectorSubcoreMesh(core_axis_name="c", subcore_axis_name="s",
                                  num_cores=NCORES, num_subcores=NSUB)

    @functools.partial(
        pl.kernel, mesh=mesh,
        compiler_params=pltpu.CompilerParams(needs_layout_passes=False),
        out_type=jax.ShapeDtypeStruct((nvpad, D), jnp.float32),
        scratch_types=[
            pltpu.VMEM_SHARED((2 * 3 * NSUB * lcap,), jnp.int32),
            pltpu.VMEM_SHARED((2 * NSUB * 16,), jnp.int32),
            pltpu.VMEM((arows, 2 * D), jnp.float32),
            pltpu.VMEM((2 * seg,), jnp.int32),
            pltpu.VMEM((2 * seg,), jnp.int32),
            pltpu.VMEM((lcap,), jnp.int32),
            pltpu.VMEM((lcap,), jnp.int32),
            pltpu.VMEM((lcap,), jnp.int32),
            pltpu.VMEM((1024,), jnp.int32),
            pltpu.VMEM((1024,), jnp.int32),
            pltpu.VMEM((1024,), jnp.int32),
            pltpu.VMEM((qcap,), jnp.int32),
            pltpu.VMEM((qcap,), jnp.int32),
            pltpu.VMEM((qcap,), jnp.int32),
            pltpu.VMEM((64, D), jnp.float32),
            pltpu.VMEM((64, D), jnp.float32),
            pltpu.VMEM((32, D), jnp.float32),
            pltpu.VMEM((16,), jnp.int32),
            pltpu.VMEM((NSUB * 16,), jnp.int32),
            pltpu.SemaphoreType.DMA,
            pltpu.SemaphoreType.DMA,
            pltpu.SemaphoreType.DMA,
            pltpu.SemaphoreType.DMA,
        ],
    )
    def body(sig_hbm, u_hbm, src_hbm, dst_hbm, h_out,
             lists, counts, acc, dseg, sseg, tq, sq, dq, lt, ls, ldl,
             tq2, sq2, dq2, sigb, ub, hbuf, cntw, cbuf,
             sem1, sem2, sem3, sem4):
        cc = lax.axis_index("c")
        ss = lax.axis_index("s")
        iota = lax.iota(jnp.int32, 16)
        zv = jnp.zeros((16,), jnp.float32)

        def fire_batch(off):
            cp_s = pltpu.async_copy(
                sig_hbm.at[tq2.at[pl.ds(off, 64)]], sigb, sem1)
            cp_u = pltpu.async_copy(
                u_hbm.at[sq2.at[pl.ds(off, 64)]], ub, sem2)
            cp_s.wait()
            cp_u.wait()

            @pl.loop(0, 4)
            def _(q):
                dlv = dq2[pl.ds(off + q * 16, 16)]
                for lane in range(16):
                    rr = q * 16 + lane
                    dl = dlv[lane]
                    for k in range(8):
                        sl = pl.ds(k * 16, 16)
                        sl2 = pl.ds(D + k * 16, 16)
                        sg = sigb[rr, sl]
                        acc[dl, sl] = acc[dl, sl] + sg * ub[rr, sl]
                        acc[dl, sl2] = acc[dl, sl2] + sg

        def drain(qcnt):
            nfull = qcnt // 64

            def fire_b(b, carry):
                fire_batch(b * 64)
                return carry

            lax.fori_loop(0, nfull, fire_b, jnp.int32(0))
            rem_base = nfull * 64
            for k in range(4):
                slk = pl.ds(k * 16, 16)
                tq2[slk] = tq2[pl.ds(rem_base + k * 16, 16)]
                sq2[slk] = sq2[pl.ds(rem_base + k * 16, 16)]
                dq2[slk] = dq2[pl.ds(rem_base + k * 16, 16)]
            return qcnt - nfull * 64

        def seg_load(g):
            p = g & 1
            segbase = ss * pt + g * seg
            return (pltpu.async_copy(dst_hbm.at[pl.ds(segbase, seg)],
                                     dseg.at[pl.ds(p * seg, seg)], sem3),
                    pltpu.async_copy(src_hbm.at[pl.ds(segbase, seg)],
                                     sseg.at[pl.ds(p * seg, seg)], sem4))

        @pl.loop(0, rounds)
        def _(r):
            base = (2 * r + cc) * ch

            @pl.loop(0, arows)
            def _(rw):
                for k in range(16):
                    acc[rw, pl.ds(k * 16, 16)] = zv

            seg_load(0)

            def seg_body(g, qcnt):
                segbase = ss * pt + g * seg
                gp = g & 1
                pltpu.make_async_copy(dst_hbm.at[pl.ds(segbase, seg)],
                                      dseg.at[pl.ds(gp * seg, seg)],
                                      sem3).wait()
                pltpu.make_async_copy(src_hbm.at[pl.ds(segbase, seg)],
                                      sseg.at[pl.ds(gp * seg, seg)],
                                      sem4).wait()

                @pl.when(g + 1 < nseg)
                def _():
                    seg_load(g + 1)

                def scan_body(v, cnt):
                    sl = pl.ds(gp * seg + v * 16, 16)
                    d = dseg[sl]
                    sv = sseg[sl]
                    m = (d >= base) & (d < base + ch)
                    pos = plsc.cumsum(m.astype(jnp.int32))
                    idx = cnt + pos - 1
                    plsc.store_scatter(tq, [idx], segbase + v * 16 + iota,
                                       mask=m)
                    plsc.store_scatter(sq, [idx], sv, mask=m)
                    plsc.store_scatter(dq, [idx], d - base, mask=m)
                    return cnt + jnp.max(pos)

                cnt = lax.fori_loop(0, seg // 16, scan_body, jnp.int32(0))

                cnt64 = ((cnt + 63) // 64) * 64
                for k in range(4):
                    sp = cnt + k * 16 + iota
                    plsc.store_scatter(dq, [sp], jnp.full(
                        (16,), NSUB * stripe, jnp.int32), mask=sp < cnt64)

                par = (g & 1)
                cntw[pl.ds(0, 16)] = jnp.full((16,), 0, jnp.int32) + cnt64
                pltpu.sync_copy(
                    cntw, counts.at[pl.ds((par * NSUB + ss) * 16, 16)])
                plsc.subcore_barrier()
                pltpu.sync_copy(
                    counts.at[pl.ds(par * NSUB * 16, NSUB * 16)], cbuf)
                cnts = plsc.load_gather(cbuf, [iota * 16])
                incl = plsc.cumsum(cnts)
                excl = incl - cnts
                myoff = pl.multiple_of(
                    jnp.sum(jnp.where(iota == ss, excl, 0)), 64)
                total = jnp.max(incl)
                lb0 = par * 3 * W

                def pub(q, carry):
                    slq = pl.ds(q * 64, 64)
                    lb = lb0 + myoff + q * 64
                    pltpu.sync_copy(tq.at[slq], lists.at[pl.ds(lb, 64)])
                    pltpu.sync_copy(sq.at[slq], lists.at[pl.ds(lb + W, 64)])
                    pltpu.sync_copy(dq.at[slq],
                                    lists.at[pl.ds(lb + 2 * W, 64)])
                    return carry

                lax.fori_loop(0, cnt64 // 64, pub, jnp.int32(0))
                plsc.subcore_barrier()

                def chunk_body(q, qcnt):
                    lb = lb0 + q * 1024
                    pltpu.sync_copy(lists.at[pl.ds(lb, 1024)], lt)
                    pltpu.sync_copy(lists.at[pl.ds(lb + W, 1024)], ls)
                    pltpu.sync_copy(lists.at[pl.ds(lb + 2 * W, 1024)], ldl)
                    nvr = (jnp.minimum(total - q * 1024, 1024) + 15) // 16

                    def filt(v, qcnt):
                        sl = pl.ds(v * 16, 16)
                        tv = lt[sl]
                        sv = ls[sl]
                        dl = ldl[sl]
                        gl = q * 1024 + v * 16 + iota
                        own = (gl < total) & ((dl >> shift) == ss)
                        pos = plsc.cumsum(own.astype(jnp.int32))
                        idx = qcnt + pos - 1
                        plsc.store_scatter(tq2, [idx], tv, mask=own)
                        plsc.store_scatter(sq2, [idx], sv, mask=own)
                        plsc.store_scatter(
                            dq2, [idx], dl & (stripe - 1), mask=own)
                        return qcnt + jnp.max(pos)

                    return drain(lax.fori_loop(0, nvr, filt, qcnt))

                qcnt = lax.fori_loop(0, (total + 1023) // 1024, chunk_body,
                                     qcnt)
                return qcnt

            qcnt = lax.fori_loop(0, nseg, seg_body, jnp.int32(0))

            padtot = ((qcnt + 63) // 64) * 64
            for k in range(4):
                pidx = qcnt + k * 16 + iota
                pm = pidx < padtot
                pv = k * 16 + iota
                plsc.store_scatter(tq2, [pidx], pv, mask=pm)
                plsc.store_scatter(sq2, [pidx], pv, mask=pm)
                plsc.store_scatter(dq2, [pidx], stripe + (pv & 7), mask=pm)
            qcnt = drain(padtot)

            @pl.loop(0, stripe // 32)
            def _(o):
                off = o * 32

                @pl.loop(0, 32)
                def _(rw):
                    for k in range(8):
                        sl = pl.ds(k * 16, 16)
                        num = acc[off + rw, sl]
                        den = acc[off + rw, pl.ds(D + k * 16, 16)]
                        hbuf[rw, sl] = num / (den + 1e-6)

                pltpu.sync_copy(
                    hbuf, h_out.at[pl.ds(base + ss * stripe + off, 32)])

    return body(sig, u_t, src, dst)


def _gcao_layer(x, y, src, dst, p, ch):
    nv = x.shape[0]
    w4 = jnp.concatenate([p['W_sg'], p['W_dg'], p['W_du'], p['W_su']], axis=1)
    b4 = jnp.concatenate([p['b_sg'], p['b_dg'], p['b_du'], p['b_su']])
    a_t, b_t, u_t, s_t = _mm_multi(x, w4, b4, 4, 1000)
    c_t, = _mm_multi(y, p['W_eg'], p['b_eg'], 1, 2000)
    ge = p['g_edge'] * BN_SCALE
    gbe = jnp.concatenate([ge, p['beta_edge']])
    sig, y_out = _edge_pass(a_t, b_t, c_t, y, src, dst, gbe)
    h = _aggregate(sig, u_t, src, dst, nv, ch)
    gn = jnp.stack([p['g_node'] * BN_SCALE, p['beta_node']])
    x_out = _node_update(x, s_t, h[:nv], gn, 1000)
    return x_out, y_out


def kernel(atom_feats, bond_attr, triplet_feats, edge_index, angle_index,
           params1, params2):
    a_src, a_dst = angle_index[0], angle_index[1]
    e_src, e_dst = edge_index[0], edge_index[1]
    bond_mid, triplet_out = _gcao_layer(bond_attr, triplet_feats,
                                        a_src, a_dst, params1, ch=4096)
    atom_out, bond_out = _gcao_layer(atom_feats, bond_mid,
                                     e_src, e_dst, params2, ch=4096)
    return (atom_out, bond_out, triplet_out)
